# baseline (speedup 1.0000x reference)
import functools

import jax
import jax.numpy as jnp
from jax import lax
from jax.experimental import pallas as pl
from jax.experimental.pallas import tpu as pltpu

H = 1024
I = 256
ER = 15
TOPK = 2
TB = 512


def _silu(v):
    return v * jax.nn.sigmoid(v)


def _dotT(a, b):
    return lax.dot_general(a, b, (((1,), (1,)), ((), ())),
                           preferred_element_type=jnp.float32)


def _moe_block(x_ref, sg_ref, su_ref, sd_ref, rg_ref, ru_ref, rd_ref,
               wr_ref, rb_ref, out_ref):
    e = pl.program_id(1)
    x = x_ref[...]
    logits = _dotT(x, wr_ref[...]) + rb_ref[...]
    probs = jax.nn.sigmoid(logits)
    idx = lax.broadcasted_iota(jnp.int32, probs.shape, 1)
    v1 = jnp.max(probs, axis=1, keepdims=True)
    i1 = jnp.min(jnp.where(probs == v1, idx, ER), axis=1, keepdims=True)
    p2 = jnp.where(idx == i1, -jnp.inf, probs)
    v2 = jnp.max(p2, axis=1, keepdims=True)
    i2 = jnp.min(jnp.where(p2 == v2, idx, ER), axis=1, keepdims=True)
    we = (jnp.where(i1 == e, v1, 0.0) + jnp.where(i2 == e, v2, 0.0)) / (v1 + v2)

    ge = _dotT(x, rg_ref[0])
    ue = _dotT(x, ru_ref[0])
    he = _silu(ge) * ue * we
    contrib = _dotT(he, rd_ref[0])

    @pl.when(e == 0)
    def _init():
        g = _dotT(x, sg_ref[...])
        u = _dotT(x, su_ref[...])
        shared = _dotT(_silu(g) * u, sd_ref[...])
        out_ref[...] = shared + contrib

    @pl.when(e != 0)
    def _acc():
        out_ref[...] += contrib


def kernel(x, sg, su, sd, rg, ru, rd, Wr, rb):
    orig_shape = x.shape
    xs = x.reshape(-1, H)
    T = xs.shape[0]
    grid = (T // TB, ER)
    out = pl.pallas_call(
        _moe_block,
        grid=grid,
        in_specs=[
            pl.BlockSpec((TB, H), lambda i, e: (i, 0)),
            pl.BlockSpec((I, H), lambda i, e: (0, 0)),
            pl.BlockSpec((I, H), lambda i, e: (0, 0)),
            pl.BlockSpec((H, I), lambda i, e: (0, 0)),
            pl.BlockSpec((1, I, H), lambda i, e: (e, 0, 0)),
            pl.BlockSpec((1, I, H), lambda i, e: (e, 0, 0)),
            pl.BlockSpec((1, H, I), lambda i, e: (e, 0, 0)),
            pl.BlockSpec((ER, H), lambda i, e: (0, 0)),
            pl.BlockSpec((ER,), lambda i, e: (0,)),
        ],
        out_specs=pl.BlockSpec((TB, H), lambda i, e: (i, 0)),
        out_shape=jax.ShapeDtypeStruct((T, H), jnp.float32),
    )(xs, sg, su, sd, rg, ru, rd, Wr, rb)
    return out.reshape(orig_shape)
